# baseline (speedup 1.0000x reference)
import functools

import jax
import jax.numpy as jnp
from jax import lax
from jax.experimental import pallas as pl
from jax.experimental.pallas import tpu as pltpu
from jax.experimental.pallas import tpu_sc as plsc

N = 10000
E = 320000
G = 64
IN, HID, OUT = 128, 256, 128

L = 16
K = 80
NCHUNK = E // K
NC, NS = 2, 16
NW = NC * NS
CPT1 = NCHUNK // NW
CPT2 = NCHUNK // NS
NPAD = 10240
RPT = N // NS
EPAD = E + 16 * K

_f32 = jnp.float32
_i32 = jnp.int32


def _mesh():
  return plsc.VectorSubcoreMesh(core_axis_name="c", subcore_axis_name="s")


def _make_pass1(D):
  KD = D // L
  NHALF = CPT1 // 2
  assert CPT1 % 2 == 1

  @functools.partial(
      pl.kernel,
      out_type=(
          jax.ShapeDtypeStruct((EPAD,), _f32),
          jax.ShapeDtypeStruct((NC, NPAD), _f32),
          jax.ShapeDtypeStruct((NC, NPAD), _f32),
      ),
      mesh=_mesh(),
      compiler_params=pltpu.CompilerParams(needs_layout_passes=False),
      scratch_types=[
          pltpu.VMEM((D,), _f32),
          pltpu.VMEM((K,), _i32),
          pltpu.VMEM((K,), _i32),
          pltpu.VMEM((K,), _i32),
          pltpu.VMEM((K,), _i32),
          pltpu.VMEM((K, D), _f32),
          pltpu.VMEM((K, D), _f32),
          pltpu.VMEM((K, D), _f32),
          pltpu.VMEM((K, D), _f32),
          pltpu.VMEM((K,), _f32),
          pltpu.VMEM((K * 17,), _f32),
          pltpu.VMEM((K,), _f32),
          pltpu.VMEM((K,), _i32),
          pltpu.VMEM_SHARED((NPAD,), _f32),
          pltpu.VMEM_SHARED((NPAD,), _f32),
          pltpu.SemaphoreType.DMA,
          pltpu.SemaphoreType.DMA,
          pltpu.SemaphoreType.DMA,
          pltpu.SemaphoreType.DMA,
      ],
  )
  def pass1(xl_hbm, xr_hbm, att_hbm, src_hbm, dst_hbm, zer_hbm,
            l_out, suml_out, cnt_out,
            attv, sidxA, didxA, sidxB, didxB, xlbA, xrbA, xlbB, xrbB,
            lbuf, pacc, ones_v, didxS, suml_sp, cnt_sp,
            semA, semB, semIA, semIB):
    cid = lax.axis_index("c")
    sid = lax.axis_index("s")
    wid = sid * NC + cid

    pltpu.sync_copy(att_hbm, attv)

    @pl.when(sid == 0)
    def _():
      pltpu.sync_copy(zer_hbm, suml_sp)
      pltpu.sync_copy(zer_hbm, cnt_sp)

    def _setones(i, c):
      ones_v[pl.ds(i * L, L)] = jnp.full((L,), 1.0, _f32)
      return c

    lax.fori_loop(0, K // L, _setones, 0)
    plsc.subcore_barrier()

    row0 = wid * CPT1
    attregs = tuple(attv[pl.ds(k * L, L)] for k in range(KD))
    lane = lax.broadcasted_iota(_i32, (L,), 0)

    def idx_load(ci, sidx, didx, sem):
      base = (row0 + ci) * K
      pltpu.async_copy(src_hbm.at[pl.ds(base, K)], sidx, sem)
      pltpu.async_copy(dst_hbm.at[pl.ds(base, K)], didx, sem)

    def idx_wait(sidx, didx, sem):
      pltpu.make_async_copy(src_hbm.at[pl.ds(0, K)], sidx, sem).wait()
      pltpu.make_async_copy(dst_hbm.at[pl.ds(0, K)], didx, sem).wait()

    def gat_issue(sidx, didx, xlb, xrb, sem):
      pltpu.async_copy(xl_hbm.at[sidx], xlb, sem)
      pltpu.async_copy(xr_hbm.at[didx], xrb, sem)

    def gat_wait(sidx, didx, xlb, xrb, sem):
      pltpu.make_async_copy(xl_hbm.at[sidx], xlb, sem).wait()
      pltpu.make_async_copy(xr_hbm.at[didx], xrb, sem).wait()

    def save_didx(didx):
      def cp(g, c):
        sl = pl.ds(g * L, L)
        didxS[sl] = didx[sl]
        return c

      lax.fori_loop(0, K // L, cp, 0)

    def compute(ci, xlb, xrb, ar):
      base = (row0 + ci) * K

      def edge_body(e, a):
        acc = jnp.zeros((L,), _f32)
        for k in range(KD):
          v = xlb[e, pl.ds(k * L, L)] + xrb[e, pl.ds(k * L, L)]
          v = jnp.maximum(v, v * 0.2)
          acc = acc + v * a[k]
        pacc[pl.ds(e * 17, L)] = acc
        return a

      ar = lax.fori_loop(0, K, edge_body, ar, unroll=2)

      for g in range(K // L):
        pbase = (lane + (g * L)) * 17

        def red(r, a):
          return a + plsc.load_gather(pacc, [pbase + r])

        lbuf[pl.ds(g * L, L)] = lax.fori_loop(
            0, L, red, jnp.zeros((L,), _f32))

      pltpu.sync_copy(lbuf, l_out.at[pl.ds(base, K)])
      pltpu.sync_copy(lbuf, suml_sp.at[didxS], add=True)
      pltpu.sync_copy(ones_v, cnt_sp.at[didxS], add=True)
      return ar

    idx_load(0, sidxA, didxA, semIA)
    idx_wait(sidxA, didxA, semIA)
    gat_issue(sidxA, didxA, xlbA, xrbA, semA)
    idx_load(1, sidxB, didxB, semIB)

    def body(i, ar):
      idx_wait(sidxB, didxB, semIB)
      gat_issue(sidxB, didxB, xlbB, xrbB, semB)
      gat_wait(sidxA, didxA, xlbA, xrbA, semA)
      save_didx(didxA)
      idx_load(2 * i + 2, sidxA, didxA, semIA)
      ar = compute(2 * i, xlbA, xrbA, ar)
      idx_wait(sidxA, didxA, semIA)
      gat_issue(sidxA, didxA, xlbA, xrbA, semA)
      gat_wait(sidxB, didxB, xlbB, xrbB, semB)
      save_didx(didxB)
      idx_load(2 * i + 3, sidxB, didxB, semIB)
      ar = compute(2 * i + 1, xlbB, xrbB, ar)
      return ar

    ar = lax.fori_loop(0, NHALF, body, attregs)
    idx_wait(sidxB, didxB, semIB)
    gat_wait(sidxA, didxA, xlbA, xrbA, semA)
    save_didx(didxA)
    compute(CPT1 - 1, xlbA, xrbA, ar)

    plsc.subcore_barrier()

    @pl.when(sid == 0)
    def _():
      pltpu.sync_copy(suml_sp, suml_out.at[cid])
      pltpu.sync_copy(cnt_sp, cnt_out.at[cid])

  return pass1


def _make_pass2(D, feature_split):
  H = D // 2 if feature_split else D
  KH = H // L
  cpt = CPT2 if feature_split else CPT1
  nhalf = cpt // 2
  odd = cpt % 2 == 1

  @functools.partial(
      pl.kernel,
      out_type=(
          jax.ShapeDtypeStruct((NC, N, H), _f32),
          jax.ShapeDtypeStruct((NC, NPAD), _f32),
      ),
      mesh=_mesh(),
      compiler_params=pltpu.CompilerParams(needs_layout_passes=False),
      scratch_types=[
          pltpu.VMEM((K,), _i32),
          pltpu.VMEM((K,), _i32),
          pltpu.VMEM((K,), _i32),
          pltpu.VMEM((K,), _f32),
          pltpu.VMEM((K + L,), _f32),
          pltpu.VMEM((K, H), _f32),
          pltpu.VMEM((K,), _i32),
          pltpu.VMEM((K,), _i32),
          pltpu.VMEM((K,), _i32),
          pltpu.VMEM((K,), _f32),
          pltpu.VMEM((K + L,), _f32),
          pltpu.VMEM((K, H), _f32),
          pltpu.VMEM((K, H), _f32),
          pltpu.VMEM((NPAD,), _f32),
          pltpu.VMEM((K,), _i32),
          pltpu.VMEM((K,), _i32),
          pltpu.VMEM_SHARED((N, H), _f32),
          pltpu.VMEM_SHARED((NPAD,), _f32),
          pltpu.SemaphoreType.DMA,
          pltpu.SemaphoreType.DMA,
          pltpu.SemaphoreType.DMA,
          pltpu.SemaphoreType.DMA,
      ],
  )
  def pass2(xl2_hbm, src_hbm, dst_hbm, l_hbm, mean_hbm,
            zrow_hbm, zer_hbm,
            acc_out, den_out,
            sidxA, didxA, gidxA, lbA, exbA, xbA,
            sidxB, didxB, gidxB, lbB, exbB, xbB,
            ob, mloc, didxSA, didxSB, acc_sp, den_sp,
            semA, semB, semIA, semIB):
    cid = lax.axis_index("c")
    sid = lax.axis_index("s")

    pltpu.sync_copy(mean_hbm, mloc)
    pltpu.sync_copy(zrow_hbm, acc_sp.at[pl.ds(sid * RPT, RPT)])

    @pl.when(sid == 0)
    def _():
      pltpu.sync_copy(zer_hbm, den_sp)

    plsc.subcore_barrier()

    if feature_split:
      row0 = sid * cpt
    else:
      row0 = (sid * NC + cid) * cpt

    def idx_load(ci, sidx, didx, lb, sem):
      base = (row0 + ci) * K
      pltpu.async_copy(src_hbm.at[pl.ds(base, K)], sidx, sem)
      pltpu.async_copy(dst_hbm.at[pl.ds(base, K)], didx, sem)
      pltpu.async_copy(l_hbm.at[pl.ds(base, K)], lb, sem)

    def idx_wait(sidx, didx, lb, sem):
      pltpu.make_async_copy(src_hbm.at[pl.ds(0, K)], sidx, sem).wait()
      pltpu.make_async_copy(dst_hbm.at[pl.ds(0, K)], didx, sem).wait()
      pltpu.make_async_copy(l_hbm.at[pl.ds(0, K)], lb, sem).wait()

    def gat_issue(sidx, gidx, xb, sem):
      def grp(g, c2):
        sl = pl.ds(g * L, L)
        if feature_split:
          gidx[sl] = sidx[sl] * 2 + cid
        else:
          gidx[sl] = sidx[sl]
        return c2

      lax.fori_loop(0, K // L, grp, 0)
      pltpu.async_copy(xl2_hbm.at[gidx], xb, sem)

    def compute(didx, gidx, lb, exb, xb, didxS, sem, prefetch):
      def grp(g, c2):
        sl = pl.ds(g * L, L)
        mv = plsc.load_gather(mloc, [didx[sl]])
        exb[sl] = jnp.exp(lb[sl] - mv)
        didxS[sl] = didx[sl]
        return c2

      lax.fori_loop(0, K // L, grp, 0)
      pltpu.make_async_copy(xl2_hbm.at[gidx], xb, sem).wait()
      if prefetch is not None:
        ci, sidx2, didx2, lb2, sem2 = prefetch
        idx_load(ci, sidx2, didx2, lb2, sem2)

      if feature_split:
        @pl.when(cid == 0)
        def _():
          pltpu.sync_copy(exb.at[pl.ds(0, K)], den_sp.at[didxS], add=True)
      else:
        pltpu.sync_copy(exb.at[pl.ds(0, K)], den_sp.at[didxS], add=True)

      def wrow(e, c2):
        s = exb[pl.ds(e, L)][0]
        for k in range(KH):
          sl = pl.ds(k * L, L)
          ob[e, sl] = xb[e, sl] * s
        return c2

      lax.fori_loop(0, K, wrow, 0, unroll=2)
      pltpu.sync_copy(ob, acc_sp.at[didxS], add=True)

    idx_load(0, sidxA, didxA, lbA, semIA)
    idx_wait(sidxA, didxA, lbA, semIA)
    gat_issue(sidxA, gidxA, xbA, semA)
    idx_load(1, sidxB, didxB, lbB, semIB)

    def body(i, c):
      idx_wait(sidxB, didxB, lbB, semIB)
      gat_issue(sidxB, gidxB, xbB, semB)
      compute(didxA, gidxA, lbA, exbA, xbA, didxSA, semA,
              (2 * i + 2, sidxA, didxA, lbA, semIA))
      idx_wait(sidxA, didxA, lbA, semIA)

      nxt = 2 * i + 2
      if odd:
        gat_issue(sidxA, gidxA, xbA, semA)
      else:
        @pl.when(nxt < cpt)
        def _():
          gat_issue(sidxA, gidxA, xbA, semA)

      compute(didxB, gidxB, lbB, exbB, xbB, didxSB, semB,
              (2 * i + 3, sidxB, didxB, lbB, semIB))
      return c

    lax.fori_loop(0, nhalf, body, 0)
    idx_wait(sidxB, didxB, lbB, semIB)
    if odd:
      compute(didxA, gidxA, lbA, exbA, xbA, didxSA, semA, None)
    plsc.subcore_barrier()

    @pl.when(sid < 10)
    def _():
      pltpu.sync_copy(acc_sp.at[pl.ds(sid * 1000, 1000)],
                      acc_out.at[cid, pl.ds(sid * 1000, 1000)])

    @pl.when(sid == 0)
    def _():
      pltpu.sync_copy(den_sp, den_out.at[cid])

  return pass2


_BM = 1000


def _mean_tc(suml, cnt):
  def body(s_ref, c_ref, m_ref):
    c = c_ref[0:1] + c_ref[1:2]
    m_ref[...] = (s_ref[0:1] + s_ref[1:2]) / jnp.maximum(c, 1.0)

  out = pl.pallas_call(
      body,
      in_specs=[
          pl.BlockSpec((NC, NPAD), lambda: (0, 0)),
          pl.BlockSpec((NC, NPAD), lambda: (0, 0)),
      ],
      out_specs=pl.BlockSpec((1, NPAD), lambda: (0, 0)),
      out_shape=jax.ShapeDtypeStruct((1, NPAD), _f32),
  )(suml, cnt)
  return out.reshape(NPAD)


def _mm2(x, Wl, Wr):
  M, KD = x.shape
  Do = Wl.shape[1]

  def body(x_ref, wl_ref, wr_ref, xl_ref, xr_ref):
    xb = x_ref[...]
    xl_ref[...] = jnp.dot(xb, wl_ref[...], preferred_element_type=_f32)
    xr_ref[...] = jnp.dot(xb, wr_ref[...], preferred_element_type=_f32)

  return pl.pallas_call(
      body,
      grid=(M // _BM,),
      in_specs=[
          pl.BlockSpec((_BM, KD), lambda i: (i, 0)),
          pl.BlockSpec((KD, Do), lambda i: (0, 0)),
          pl.BlockSpec((KD, Do), lambda i: (0, 0)),
      ],
      out_specs=[
          pl.BlockSpec((_BM, Do), lambda i: (i, 0)),
          pl.BlockSpec((_BM, Do), lambda i: (i, 0)),
      ],
      out_shape=[
          jax.ShapeDtypeStruct((M, Do), _f32),
          jax.ShapeDtypeStruct((M, Do), _f32),
      ],
  )(x, Wl, Wr)


def _fixup_mm2(acc, den0, den1, b, Wl, Wr, relu):
  _, M, H = acc.shape
  Do = Wl.shape[1]
  wl0, wl1 = Wl[:H], Wl[H:]
  wr0, wr1 = Wr[:H], Wr[H:]
  b0, b1 = b[:, :H], b[:, H:]

  def body(acc_ref, d0_ref, d1_ref, b0_ref, b1_ref,
           wl0_ref, wl1_ref, wr0_ref, wr1_ref, xl_ref, xr_ref):
    d = d0_ref[...] + d1_ref[...] + 1e-16
    h0 = acc_ref[0] / d + b0_ref[...]
    h1 = acc_ref[1] / d + b1_ref[...]
    if relu:
      h0 = jnp.maximum(h0, 0.0)
      h1 = jnp.maximum(h1, 0.0)
    xl_ref[...] = (jnp.dot(h0, wl0_ref[...], preferred_element_type=_f32) +
                   jnp.dot(h1, wl1_ref[...], preferred_element_type=_f32))
    xr_ref[...] = (jnp.dot(h0, wr0_ref[...], preferred_element_type=_f32) +
                   jnp.dot(h1, wr1_ref[...], preferred_element_type=_f32))

  wspec = pl.BlockSpec((H, Do), lambda i: (0, 0))
  bspec = pl.BlockSpec((1, H), lambda i: (0, 0))
  return pl.pallas_call(
      body,
      grid=(M // _BM,),
      in_specs=[
          pl.BlockSpec((2, _BM, H), lambda i: (0, i, 0)),
          pl.BlockSpec((_BM, 1), lambda i: (i, 0)),
          pl.BlockSpec((_BM, 1), lambda i: (i, 0)),
          bspec, bspec, wspec, wspec, wspec, wspec,
      ],
      out_specs=[
          pl.BlockSpec((_BM, Do), lambda i: (i, 0)),
          pl.BlockSpec((_BM, Do), lambda i: (i, 0)),
      ],
      out_shape=[
          jax.ShapeDtypeStruct((M, Do), _f32),
          jax.ShapeDtypeStruct((M, Do), _f32),
      ],
  )(acc, den0, den1, b0, b1, wl0, wl1, wr0, wr1)


def _final_pool(acc, den0, den1, b, batch):
  _, M, H = acc.shape
  nblk = M // _BM

  def body(acc_ref, d0_ref, d1_ref, b_ref, batch_ref, out_ref, sums, cnts):
    i = pl.program_id(0)

    @pl.when(i == 0)
    def _():
      sums[...] = jnp.zeros_like(sums)
      cnts[...] = jnp.zeros_like(cnts)

    d = d0_ref[...] + d1_ref[...] + 1e-16
    h = (acc_ref[0] + acc_ref[1]) / d + b_ref[...]
    ids = lax.broadcasted_iota(_i32, (1, G), 1)
    oh = (batch_ref[...] == ids).astype(_f32)
    sums[...] += lax.dot_general(oh, h, (((0,), (0,)), ((), ())),
                                 preferred_element_type=_f32)
    cnts[...] += lax.dot_general(oh, jnp.ones_like(h),
                                 (((0,), (0,)), ((), ())),
                                 preferred_element_type=_f32)

    @pl.when(i == nblk - 1)
    def _():
      out_ref[...] = sums[...] / jnp.maximum(cnts[...], 1.0)

  bspec = pl.BlockSpec((1, H), lambda i: (0, 0))
  dspec = pl.BlockSpec((_BM, 1), lambda i: (i, 0))
  return pl.pallas_call(
      body,
      grid=(nblk,),
      in_specs=[
          pl.BlockSpec((2, _BM, H), lambda i: (0, i, 0)),
          dspec, dspec, bspec, dspec,
      ],
      out_specs=pl.BlockSpec((G, H), lambda i: (0, 0)),
      out_shape=jax.ShapeDtypeStruct((G, H), _f32),
      scratch_shapes=[
          pltpu.VMEM((G, H), _f32),
          pltpu.VMEM((G, H), _f32),
      ],
  )(acc, den0, den1, b, batch)


_pass1_256 = _make_pass1(256)
_pass1_128 = _make_pass1(128)
_pass2_256 = _make_pass2(256, feature_split=True)
_pass2_128 = _make_pass2(128, feature_split=False)


def kernel(x, edge_index, batch, Wl1, Wr1, att1, b1,
           Wl2, Wr2, att2, b2, Wl3, Wr3, att3, b3):
  src1 = jnp.pad(edge_index[0].astype(_i32), (0, EPAD - E))
  dst1 = jnp.pad(edge_index[1].astype(_i32), (0, EPAD - E))
  zer = jnp.zeros((NPAD,), _f32)
  zrow = jnp.zeros((RPT, 128), _f32)

  def gat(xl, xr, att, D):
    p1 = _pass1_256 if D == 256 else _pass1_128
    p2 = _pass2_256 if D == 256 else _pass2_128
    l, suml, cnt = p1(xl, xr, att, src1, dst1, zer)
    mean = _mean_tc(suml, cnt)
    table = xl.reshape(2 * N, D // 2) if D == 256 else xl
    acc, den = p2(table, src1, dst1, l, mean, zrow, zer)
    d0 = den[0, :N].reshape(N, 1)
    d1 = den[1, :N].reshape(N, 1)
    return acc, d0, d1

  xl1, xr1 = _mm2(x, Wl1, Wr1)
  acc1, d10, d11 = gat(xl1, xr1, att1, 256)
  xl2, xr2 = _fixup_mm2(acc1, d10, d11, b1.reshape(1, -1), Wl2, Wr2,
                        relu=True)
  acc2, d20, d21 = gat(xl2, xr2, att2, 256)
  xl3, xr3 = _fixup_mm2(acc2, d20, d21, b2.reshape(1, -1), Wl3, Wr3,
                        relu=True)
  acc3, d30, d31 = gat(xl3, xr3, att3, 128)
  return _final_pool(acc3, d30, d31, b3.reshape(1, -1), batch.reshape(N, 1))


# speedup vs baseline: 1.7061x; 1.7061x over previous
import functools

import jax
import jax.numpy as jnp
from jax import lax
from jax.experimental import pallas as pl
from jax.experimental.pallas import tpu as pltpu
from jax.experimental.pallas import tpu_sc as plsc

N = 10000
E = 320000
G = 64
IN, HID, OUT = 128, 256, 128

L = 16
K = 80
NCHUNK = E // K
NC, NS = 2, 16
NW = NC * NS
CPT1 = NCHUNK // NW
CPT2 = NCHUNK // NS
NPAD = 10240
RPT = N // NS
EPAD = E + 16 * K

_f32 = jnp.float32
_i32 = jnp.int32


def _mesh():
  return plsc.VectorSubcoreMesh(core_axis_name="c", subcore_axis_name="s")


def _make_pass1(D):
  KD = D // L
  NHALF = CPT1 // 2
  assert CPT1 % 2 == 1

  @functools.partial(
      pl.kernel,
      out_type=(
          jax.ShapeDtypeStruct((EPAD,), _f32),
          jax.ShapeDtypeStruct((NC, NPAD), _f32),
          jax.ShapeDtypeStruct((NC, NPAD), _f32),
      ),
      mesh=_mesh(),
      compiler_params=pltpu.CompilerParams(needs_layout_passes=False),
      scratch_types=[
          pltpu.VMEM((D,), _f32),
          pltpu.VMEM((K,), _i32),
          pltpu.VMEM((K,), _i32),
          pltpu.VMEM((K,), _i32),
          pltpu.VMEM((K,), _i32),
          pltpu.VMEM((K, D), _f32),
          pltpu.VMEM((K, D), _f32),
          pltpu.VMEM((K, D), _f32),
          pltpu.VMEM((K, D), _f32),
          pltpu.VMEM((K,), _f32),
          pltpu.VMEM((K * 17,), _f32),
          pltpu.VMEM((K,), _f32),
          pltpu.VMEM((K,), _i32),
          pltpu.VMEM_SHARED((NPAD,), _f32),
          pltpu.VMEM_SHARED((NPAD,), _f32),
          pltpu.SemaphoreType.DMA,
          pltpu.SemaphoreType.DMA,
          pltpu.SemaphoreType.DMA,
          pltpu.SemaphoreType.DMA,
      ],
  )
  def pass1(xl_hbm, xr_hbm, att_hbm, src_hbm, dst_hbm, zer_hbm,
            l_out, suml_out, cnt_out,
            attv, sidxA, didxA, sidxB, didxB, xlbA, xrbA, xlbB, xrbB,
            lbuf, pacc, ones_v, didxS, suml_sp, cnt_sp,
            semA, semB, semIA, semIB):
    cid = lax.axis_index("c")
    sid = lax.axis_index("s")
    wid = sid * NC + cid

    pltpu.sync_copy(att_hbm, attv)

    @pl.when(sid == 0)
    def _():
      pltpu.sync_copy(zer_hbm, suml_sp)
      pltpu.sync_copy(zer_hbm, cnt_sp)

    def _setones(i, c):
      ones_v[pl.ds(i * L, L)] = jnp.full((L,), 1.0, _f32)
      return c

    lax.fori_loop(0, K // L, _setones, 0)
    plsc.subcore_barrier()

    row0 = wid * CPT1
    attregs = tuple(attv[pl.ds(k * L, L)] for k in range(KD))
    lane = lax.broadcasted_iota(_i32, (L,), 0)

    def idx_load(ci, sidx, didx, sem):
      base = (row0 + ci) * K
      pltpu.async_copy(src_hbm.at[pl.ds(base, K)], sidx, sem)
      pltpu.async_copy(dst_hbm.at[pl.ds(base, K)], didx, sem)

    def idx_wait(sidx, didx, sem):
      pltpu.make_async_copy(src_hbm.at[pl.ds(0, K)], sidx, sem).wait()
      pltpu.make_async_copy(dst_hbm.at[pl.ds(0, K)], didx, sem).wait()

    def gat_issue(sidx, didx, xlb, xrb, sem):
      pltpu.async_copy(xl_hbm.at[sidx], xlb, sem)
      pltpu.async_copy(xr_hbm.at[didx], xrb, sem)

    def gat_wait(sidx, didx, xlb, xrb, sem):
      pltpu.make_async_copy(xl_hbm.at[sidx], xlb, sem).wait()
      pltpu.make_async_copy(xr_hbm.at[didx], xrb, sem).wait()

    def save_didx(didx):
      def cp(g, c):
        sl = pl.ds(g * L, L)
        didxS[sl] = didx[sl]
        return c

      lax.fori_loop(0, K // L, cp, 0)

    def compute(ci, xlb, xrb, ar):
      base = (row0 + ci) * K

      def edge_body(e, a):
        acc = jnp.zeros((L,), _f32)
        for k in range(KD):
          v = xlb[e, pl.ds(k * L, L)] + xrb[e, pl.ds(k * L, L)]
          v = jnp.maximum(v, v * 0.2)
          acc = acc + v * a[k]
        pacc[pl.ds(e * 17, L)] = acc
        return a

      ar = lax.fori_loop(0, K, edge_body, ar)

      for g in range(K // L):
        pbase = (lane + (g * L)) * 17

        def red(r, a):
          return a + plsc.load_gather(pacc, [pbase + r])

        lbuf[pl.ds(g * L, L)] = lax.fori_loop(
            0, L, red, jnp.zeros((L,), _f32))

      pltpu.sync_copy(lbuf, l_out.at[pl.ds(base, K)])
      pltpu.sync_copy(lbuf, suml_sp.at[didxS], add=True)
      pltpu.sync_copy(ones_v, cnt_sp.at[didxS], add=True)
      return ar

    idx_load(0, sidxA, didxA, semIA)
    idx_wait(sidxA, didxA, semIA)
    gat_issue(sidxA, didxA, xlbA, xrbA, semA)
    idx_load(1, sidxB, didxB, semIB)

    def body(i, ar):
      idx_wait(sidxB, didxB, semIB)
      gat_issue(sidxB, didxB, xlbB, xrbB, semB)
      gat_wait(sidxA, didxA, xlbA, xrbA, semA)
      save_didx(didxA)
      idx_load(2 * i + 2, sidxA, didxA, semIA)
      ar = compute(2 * i, xlbA, xrbA, ar)
      idx_wait(sidxA, didxA, semIA)
      gat_issue(sidxA, didxA, xlbA, xrbA, semA)
      gat_wait(sidxB, didxB, xlbB, xrbB, semB)
      save_didx(didxB)
      idx_load(2 * i + 3, sidxB, didxB, semIB)
      ar = compute(2 * i + 1, xlbB, xrbB, ar)
      return ar

    ar = lax.fori_loop(0, NHALF, body, attregs)
    idx_wait(sidxB, didxB, semIB)
    gat_wait(sidxA, didxA, xlbA, xrbA, semA)
    save_didx(didxA)
    compute(CPT1 - 1, xlbA, xrbA, ar)

    plsc.subcore_barrier()

    @pl.when(sid == 0)
    def _():
      pltpu.sync_copy(suml_sp, suml_out.at[cid])
      pltpu.sync_copy(cnt_sp, cnt_out.at[cid])

  return pass1


def _make_pass2(D, feature_split):
  H = D // 2 if feature_split else D
  KH = H // L
  cpt = CPT2 if feature_split else CPT1
  nhalf = cpt // 2
  odd = cpt % 2 == 1

  @functools.partial(
      pl.kernel,
      out_type=(
          jax.ShapeDtypeStruct((NC, N, H), _f32),
          jax.ShapeDtypeStruct((NC, NPAD), _f32),
      ),
      mesh=_mesh(),
      compiler_params=pltpu.CompilerParams(needs_layout_passes=False),
      scratch_types=[
          pltpu.VMEM((K,), _i32),
          pltpu.VMEM((K,), _i32),
          pltpu.VMEM((K,), _i32),
          pltpu.VMEM((K,), _f32),
          pltpu.VMEM((K + L,), _f32),
          pltpu.VMEM((K, H), _f32),
          pltpu.VMEM((K,), _i32),
          pltpu.VMEM((K,), _i32),
          pltpu.VMEM((K,), _i32),
          pltpu.VMEM((K,), _f32),
          pltpu.VMEM((K + L,), _f32),
          pltpu.VMEM((K, H), _f32),
          pltpu.VMEM((K, H), _f32),
          pltpu.VMEM((NPAD,), _f32),
          pltpu.VMEM((K,), _i32),
          pltpu.VMEM((K,), _i32),
          pltpu.VMEM_SHARED((N, H), _f32),
          pltpu.VMEM_SHARED((NPAD,), _f32),
          pltpu.SemaphoreType.DMA,
          pltpu.SemaphoreType.DMA,
          pltpu.SemaphoreType.DMA,
          pltpu.SemaphoreType.DMA,
      ],
  )
  def pass2(xl2_hbm, src_hbm, dst_hbm, l_hbm, mean_hbm,
            zrow_hbm, zer_hbm,
            acc_out, den_out,
            sidxA, didxA, gidxA, lbA, exbA, xbA,
            sidxB, didxB, gidxB, lbB, exbB, xbB,
            ob, mloc, didxSA, didxSB, acc_sp, den_sp,
            semA, semB, semIA, semIB):
    cid = lax.axis_index("c")
    sid = lax.axis_index("s")

    pltpu.sync_copy(mean_hbm, mloc)
    pltpu.sync_copy(zrow_hbm, acc_sp.at[pl.ds(sid * RPT, RPT)])

    @pl.when(sid == 0)
    def _():
      pltpu.sync_copy(zer_hbm, den_sp)

    plsc.subcore_barrier()

    if feature_split:
      row0 = sid * cpt
    else:
      row0 = (sid * NC + cid) * cpt

    def idx_load(ci, sidx, didx, lb, sem):
      base = (row0 + ci) * K
      pltpu.async_copy(src_hbm.at[pl.ds(base, K)], sidx, sem)
      pltpu.async_copy(dst_hbm.at[pl.ds(base, K)], didx, sem)
      pltpu.async_copy(l_hbm.at[pl.ds(base, K)], lb, sem)

    def idx_wait(sidx, didx, lb, sem):
      pltpu.make_async_copy(src_hbm.at[pl.ds(0, K)], sidx, sem).wait()
      pltpu.make_async_copy(dst_hbm.at[pl.ds(0, K)], didx, sem).wait()
      pltpu.make_async_copy(l_hbm.at[pl.ds(0, K)], lb, sem).wait()

    def gat_issue(sidx, gidx, xb, sem):
      def grp(g, c2):
        sl = pl.ds(g * L, L)
        if feature_split:
          gidx[sl] = sidx[sl] * 2 + cid
        else:
          gidx[sl] = sidx[sl]
        return c2

      lax.fori_loop(0, K // L, grp, 0)
      pltpu.async_copy(xl2_hbm.at[gidx], xb, sem)

    def compute(didx, gidx, lb, exb, xb, didxS, sem, prefetch):
      def grp(g, c2):
        sl = pl.ds(g * L, L)
        mv = plsc.load_gather(mloc, [didx[sl]])
        exb[sl] = jnp.exp(lb[sl] - mv)
        didxS[sl] = didx[sl]
        return c2

      lax.fori_loop(0, K // L, grp, 0)
      pltpu.make_async_copy(xl2_hbm.at[gidx], xb, sem).wait()
      if prefetch is not None:
        ci, sidx2, didx2, lb2, sem2 = prefetch
        idx_load(ci, sidx2, didx2, lb2, sem2)

      if feature_split:
        @pl.when(cid == 0)
        def _():
          pltpu.sync_copy(exb.at[pl.ds(0, K)], den_sp.at[didxS], add=True)
      else:
        pltpu.sync_copy(exb.at[pl.ds(0, K)], den_sp.at[didxS], add=True)

      def wrow(e, c2):
        s = exb[pl.ds(e, L)][0]
        for k in range(KH):
          sl = pl.ds(k * L, L)
          ob[e, sl] = xb[e, sl] * s
        return c2

      lax.fori_loop(0, K, wrow, 0)
      pltpu.sync_copy(ob, acc_sp.at[didxS], add=True)

    idx_load(0, sidxA, didxA, lbA, semIA)
    idx_wait(sidxA, didxA, lbA, semIA)
    gat_issue(sidxA, gidxA, xbA, semA)
    idx_load(1, sidxB, didxB, lbB, semIB)

    def body(i, c):
      idx_wait(sidxB, didxB, lbB, semIB)
      gat_issue(sidxB, gidxB, xbB, semB)
      compute(didxA, gidxA, lbA, exbA, xbA, didxSA, semA,
              (2 * i + 2, sidxA, didxA, lbA, semIA))
      idx_wait(sidxA, didxA, lbA, semIA)

      nxt = 2 * i + 2
      if odd:
        gat_issue(sidxA, gidxA, xbA, semA)
      else:
        @pl.when(nxt < cpt)
        def _():
          gat_issue(sidxA, gidxA, xbA, semA)

      compute(didxB, gidxB, lbB, exbB, xbB, didxSB, semB,
              (2 * i + 3, sidxB, didxB, lbB, semIB))
      return c

    lax.fori_loop(0, nhalf, body, 0)
    idx_wait(sidxB, didxB, lbB, semIB)
    if odd:
      compute(didxA, gidxA, lbA, exbA, xbA, didxSA, semA, None)
    plsc.subcore_barrier()

    @pl.when(sid < 10)
    def _():
      pltpu.sync_copy(acc_sp.at[pl.ds(sid * 1000, 1000)],
                      acc_out.at[cid, pl.ds(sid * 1000, 1000)])

    @pl.when(sid == 0)
    def _():
      pltpu.sync_copy(den_sp, den_out.at[cid])

  return pass2


_BM = 1000


def _mean_tc(suml, cnt):
  def body(s_ref, c_ref, m_ref):
    c = c_ref[0:1] + c_ref[1:2]
    m_ref[...] = (s_ref[0:1] + s_ref[1:2]) / jnp.maximum(c, 1.0)

  out = pl.pallas_call(
      body,
      in_specs=[
          pl.BlockSpec((NC, NPAD), lambda: (0, 0)),
          pl.BlockSpec((NC, NPAD), lambda: (0, 0)),
      ],
      out_specs=pl.BlockSpec((1, NPAD), lambda: (0, 0)),
      out_shape=jax.ShapeDtypeStruct((1, NPAD), _f32),
  )(suml, cnt)
  return out.reshape(NPAD)


def _mm2(x, Wl, Wr):
  M, KD = x.shape
  Do = Wl.shape[1]

  def body(x_ref, wl_ref, wr_ref, xl_ref, xr_ref):
    xb = x_ref[...]
    xl_ref[...] = jnp.dot(xb, wl_ref[...], preferred_element_type=_f32)
    xr_ref[...] = jnp.dot(xb, wr_ref[...], preferred_element_type=_f32)

  return pl.pallas_call(
      body,
      grid=(M // _BM,),
      in_specs=[
          pl.BlockSpec((_BM, KD), lambda i: (i, 0)),
          pl.BlockSpec((KD, Do), lambda i: (0, 0)),
          pl.BlockSpec((KD, Do), lambda i: (0, 0)),
      ],
      out_specs=[
          pl.BlockSpec((_BM, Do), lambda i: (i, 0)),
          pl.BlockSpec((_BM, Do), lambda i: (i, 0)),
      ],
      out_shape=[
          jax.ShapeDtypeStruct((M, Do), _f32),
          jax.ShapeDtypeStruct((M, Do), _f32),
      ],
  )(x, Wl, Wr)


def _fixup_mm2(acc, den0, den1, b, Wl, Wr, relu):
  _, M, H = acc.shape
  Do = Wl.shape[1]
  wl0, wl1 = Wl[:H], Wl[H:]
  wr0, wr1 = Wr[:H], Wr[H:]
  b0, b1 = b[:, :H], b[:, H:]

  def body(acc_ref, d0_ref, d1_ref, b0_ref, b1_ref,
           wl0_ref, wl1_ref, wr0_ref, wr1_ref, xl_ref, xr_ref):
    d = d0_ref[...] + d1_ref[...] + 1e-16
    h0 = acc_ref[0] / d + b0_ref[...]
    h1 = acc_ref[1] / d + b1_ref[...]
    if relu:
      h0 = jnp.maximum(h0, 0.0)
      h1 = jnp.maximum(h1, 0.0)
    xl_ref[...] = (jnp.dot(h0, wl0_ref[...], preferred_element_type=_f32) +
                   jnp.dot(h1, wl1_ref[...], preferred_element_type=_f32))
    xr_ref[...] = (jnp.dot(h0, wr0_ref[...], preferred_element_type=_f32) +
                   jnp.dot(h1, wr1_ref[...], preferred_element_type=_f32))

  wspec = pl.BlockSpec((H, Do), lambda i: (0, 0))
  bspec = pl.BlockSpec((1, H), lambda i: (0, 0))
  return pl.pallas_call(
      body,
      grid=(M // _BM,),
      in_specs=[
          pl.BlockSpec((2, _BM, H), lambda i: (0, i, 0)),
          pl.BlockSpec((_BM, 1), lambda i: (i, 0)),
          pl.BlockSpec((_BM, 1), lambda i: (i, 0)),
          bspec, bspec, wspec, wspec, wspec, wspec,
      ],
      out_specs=[
          pl.BlockSpec((_BM, Do), lambda i: (i, 0)),
          pl.BlockSpec((_BM, Do), lambda i: (i, 0)),
      ],
      out_shape=[
          jax.ShapeDtypeStruct((M, Do), _f32),
          jax.ShapeDtypeStruct((M, Do), _f32),
      ],
  )(acc, den0, den1, b0, b1, wl0, wl1, wr0, wr1)


def _final_pool(acc, den0, den1, b, batch):
  _, M, H = acc.shape
  nblk = M // _BM

  def body(acc_ref, d0_ref, d1_ref, b_ref, batch_ref, out_ref, sums, cnts):
    i = pl.program_id(0)

    @pl.when(i == 0)
    def _():
      sums[...] = jnp.zeros_like(sums)
      cnts[...] = jnp.zeros_like(cnts)

    d = d0_ref[...] + d1_ref[...] + 1e-16
    h = (acc_ref[0] + acc_ref[1]) / d + b_ref[...]
    ids = lax.broadcasted_iota(_i32, (1, G), 1)
    oh = (batch_ref[...] == ids).astype(_f32)
    sums[...] += lax.dot_general(oh, h, (((0,), (0,)), ((), ())),
                                 preferred_element_type=_f32)
    cnts[...] += lax.dot_general(oh, jnp.ones_like(h),
                                 (((0,), (0,)), ((), ())),
                                 preferred_element_type=_f32)

    @pl.when(i == nblk - 1)
    def _():
      out_ref[...] = sums[...] / jnp.maximum(cnts[...], 1.0)

  bspec = pl.BlockSpec((1, H), lambda i: (0, 0))
  dspec = pl.BlockSpec((_BM, 1), lambda i: (i, 0))
  return pl.pallas_call(
      body,
      grid=(nblk,),
      in_specs=[
          pl.BlockSpec((2, _BM, H), lambda i: (0, i, 0)),
          dspec, dspec, bspec, dspec,
      ],
      out_specs=pl.BlockSpec((G, H), lambda i: (0, 0)),
      out_shape=jax.ShapeDtypeStruct((G, H), _f32),
      scratch_shapes=[
          pltpu.VMEM((G, H), _f32),
          pltpu.VMEM((G, H), _f32),
      ],
  )(acc, den0, den1, b, batch)


_pass1_256 = _make_pass1(256)
_pass1_128 = _make_pass1(128)
_pass2_256 = _make_pass2(256, feature_split=True)
_pass2_128 = _make_pass2(128, feature_split=False)


def kernel(x, edge_index, batch, Wl1, Wr1, att1, b1,
           Wl2, Wr2, att2, b2, Wl3, Wr3, att3, b3):
  src1 = jnp.pad(edge_index[0].astype(_i32), (0, EPAD - E))
  dst1 = jnp.pad(edge_index[1].astype(_i32), (0, EPAD - E))
  zer = jnp.zeros((NPAD,), _f32)
  zrow = jnp.zeros((RPT, 128), _f32)

  def gat(xl, xr, att, D):
    p1 = _pass1_256 if D == 256 else _pass1_128
    p2 = _pass2_256 if D == 256 else _pass2_128
    l, suml, cnt = p1(xl, xr, att, src1, dst1, zer)
    mean = _mean_tc(suml, cnt)
    table = xl.reshape(2 * N, D // 2) if D == 256 else xl
    acc, den = p2(table, src1, dst1, l, mean, zrow, zer)
    d0 = den[0, :N].reshape(N, 1)
    d1 = den[1, :N].reshape(N, 1)
    return acc, d0, d1

  xl1, xr1 = _mm2(x, Wl1, Wr1)
  acc1, d10, d11 = gat(xl1, xr1, att1, 256)
  xl2, xr2 = _fixup_mm2(acc1, d10, d11, b1.reshape(1, -1), Wl2, Wr2,
                        relu=True)
  acc2, d20, d21 = gat(xl2, xr2, att2, 256)
  xl3, xr3 = _fixup_mm2(acc2, d20, d21, b2.reshape(1, -1), Wl3, Wr3,
                        relu=True)
  acc3, d30, d31 = gat(xl3, xr3, att3, 128)
  return _final_pool(acc3, d30, d31, b3.reshape(1, -1), batch.reshape(N, 1))
